# baseline (speedup 1.0000x reference)
import functools

import jax
import jax.numpy as jnp
from jax import lax
from jax.experimental import pallas as pl
from jax.experimental.pallas import tpu as pltpu
from jax.experimental.pallas import tpu_sc as plsc

N = 10000
E = 160000
D = 256
HALF = 128

TILES = 16
CHUNK = 120
NCHUNK = 84
EP = TILES * NCHUNK * CHUNK
NP = 10112
RPT = NP // TILES


def _sc_agg_body(h2, src2, dstr, zrow,
                 agg_out,
                 accf,
                 src_stage,
                 idA0, idA1, idB0, idB1,
                 rows0, rows1,
                 i0, i1, i2, i3,
                 gsem0, gsem1):
    c = lax.axis_index("c")
    s = lax.axis_index("s")

    pltpu.sync_copy(src2.at[c, s], src_stage)

    pltpu.sync_copy(zrow, rows0)
    for t in range(RPT // CHUNK):
        pltpu.sync_copy(rows0, accf.at[pl.ds(s * RPT + t * CHUNK, CHUNK)])
    pltpu.sync_copy(rows0.at[pl.ds(0, RPT % CHUNK)],
                    accf.at[pl.ds(s * RPT + RPT - RPT % CHUNK, RPT % CHUNK)])
    plsc.subcore_barrier()

    g0p = pltpu.async_copy(h2.at[src_stage.at[0]], rows0, gsem0)
    g1p = pltpu.async_copy(h2.at[src_stage.at[1]], rows1, gsem1)

    def _body(j, carry):
        q0 = j * 4
        q1 = q0 + 1
        q2 = q0 + 2
        q3 = q0 + 3
        qn = lax.min(q0 + 4, NCHUNK - 2)
        cD0 = pltpu.async_copy(dstr.at[s, q0], idA0, i0)
        cD1 = pltpu.async_copy(dstr.at[s, q1], idA1, i1)
        cD2 = pltpu.async_copy(dstr.at[s, q2], idB0, i2)
        cD3 = pltpu.async_copy(dstr.at[s, q3], idB1, i3)
        pltpu.make_async_copy(h2.at[src_stage.at[q0]], rows0, gsem0).wait()
        cD0.wait()
        pltpu.sync_copy(rows0, accf.at[idA0], add=True)
        g2 = pltpu.async_copy(h2.at[src_stage.at[q2]], rows0, gsem0)
        pltpu.make_async_copy(h2.at[src_stage.at[q1]], rows1, gsem1).wait()
        cD1.wait()
        pltpu.sync_copy(rows1, accf.at[idA1], add=True)
        g3 = pltpu.async_copy(h2.at[src_stage.at[q3]], rows1, gsem1)
        g2.wait()
        cD2.wait()
        pltpu.sync_copy(rows0, accf.at[idB0], add=True)
        pltpu.async_copy(h2.at[src_stage.at[qn]], rows0, gsem0)
        g3.wait()
        cD3.wait()
        pltpu.sync_copy(rows1, accf.at[idB1], add=True)
        pltpu.async_copy(h2.at[src_stage.at[qn + 1]], rows1, gsem1)
        return carry

    lax.fori_loop(0, NCHUNK // 4, _body, 0)

    pltpu.make_async_copy(h2.at[src_stage.at[NCHUNK - 2]], rows0, gsem0).wait()
    pltpu.make_async_copy(h2.at[src_stage.at[NCHUNK - 1]], rows1, gsem1).wait()
    plsc.subcore_barrier()

    pltpu.sync_copy(accf.at[pl.ds(s * RPT, RPT)],
                    agg_out.at[c, pl.ds(s * RPT, RPT)])


@functools.cache
def _make_sc_agg():
    return pl.kernel(
        _sc_agg_body,
        mesh=plsc.VectorSubcoreMesh(core_axis_name="c", subcore_axis_name="s"),
        out_type=jax.ShapeDtypeStruct((2, NP, HALF), jnp.float32),
        scratch_types=(
            [pltpu.VMEM_SHARED((NP, HALF), jnp.float32)]
            + [pltpu.VMEM((NCHUNK, CHUNK), jnp.int32)]
            + [pltpu.VMEM((CHUNK,), jnp.int32) for _ in range(4)]
            + [pltpu.VMEM((CHUNK, HALF), jnp.float32) for _ in range(2)]
            + [pltpu.SemaphoreType.DMA for _ in range(6)]
        ),
    )


def _sc_cnt_body(dstr, zcnt, ones_h,
                 cnt_out,
                 accc,
                 idA, idB, ones_b,
                 c0, c1):
    c = lax.axis_index("c")
    s = lax.axis_index("s")
    half = NCHUNK // 2
    base = c * half

    pltpu.async_copy(dstr.at[s, base], idA, c0)
    pltpu.async_copy(dstr.at[s, base + 1], idB, c1)
    pltpu.sync_copy(ones_h, ones_b)
    pltpu.sync_copy(zcnt, accc.at[pl.ds(s * RPT, RPT)])
    plsc.subcore_barrier()

    def _body(j, carry):
        q = base + j * 2
        qn = lax.min(q + 2, base + half - 2)
        pltpu.make_async_copy(dstr.at[s, q], idA, c0).wait()
        pltpu.sync_copy(ones_b, accc.at[idA], add=True)
        pltpu.async_copy(dstr.at[s, qn], idA, c0)
        pltpu.make_async_copy(dstr.at[s, q + 1], idB, c1).wait()
        pltpu.sync_copy(ones_b, accc.at[idB], add=True)
        pltpu.async_copy(dstr.at[s, qn + 1], idB, c1)
        return carry

    lax.fori_loop(0, NCHUNK // 2 // 2, _body, 0)
    pltpu.make_async_copy(dstr.at[s, base + half - 2], idA, c0).wait()
    pltpu.make_async_copy(dstr.at[s, base + half - 1], idB, c1).wait()
    plsc.subcore_barrier()

    pltpu.sync_copy(accc.at[pl.ds(s * RPT, RPT)],
                    cnt_out.at[c, pl.ds(s * RPT, RPT)])


@functools.cache
def _make_sc_cnt():
    return pl.kernel(
        _sc_cnt_body,
        mesh=plsc.VectorSubcoreMesh(core_axis_name="c", subcore_axis_name="s"),
        out_type=jax.ShapeDtypeStruct((2, NP, HALF), jnp.float32),
        scratch_types=[
            pltpu.VMEM_SHARED((NP, HALF), jnp.float32),
            pltpu.VMEM((CHUNK,), jnp.int32),
            pltpu.VMEM((CHUNK,), jnp.int32),
            pltpu.VMEM((CHUNK, HALF), jnp.float32),
            pltpu.SemaphoreType.DMA,
            pltpu.SemaphoreType.DMA,
        ],
    )


BLK = 400


def _layer_body(h, a0, a1, cnt, wl0, wl1, wr, bl, o):
    inv = 1.0 / jnp.maximum(cnt[...], 1.0)
    acc = jnp.dot(a0[...] * inv, wl0[...], preferred_element_type=jnp.float32)
    acc += jnp.dot(a1[...] * inv, wl1[...], preferred_element_type=jnp.float32)
    acc += jnp.dot(h[...], wr[...], preferred_element_type=jnp.float32)
    o[...] = jnp.maximum(acc + bl[...], 0.0)


def _final_body(h, a0, a1, cnt, wl0, wl1, wr, bl, w1, b1, w2, b2, o):
    inv = 1.0 / jnp.maximum(cnt[...], 1.0)
    acc = jnp.dot(a0[...] * inv, wl0[...], preferred_element_type=jnp.float32)
    acc += jnp.dot(a1[...] * inv, wl1[...], preferred_element_type=jnp.float32)
    acc += jnp.dot(h[...], wr[...], preferred_element_type=jnp.float32)
    t = jnp.maximum(acc + bl[...], 0.0)
    u = jnp.dot(t, w1[...], preferred_element_type=jnp.float32) + b1[...]
    v = jnp.dot(u, w2[...], preferred_element_type=jnp.float32) + b2[...]
    m = jnp.max(v, axis=1, keepdims=True)
    lse = jnp.log(jnp.sum(jnp.exp(v - m), axis=1, keepdims=True)) + m
    o[...] = v - lse


def _row_spec(w):
    return pl.BlockSpec((BLK, w), lambda i: (i, 0))


def _full_spec(r, c):
    return pl.BlockSpec((r, c), lambda i: (0, 0))


_layer_call = pl.pallas_call(
    _layer_body,
    grid=(N // BLK,),
    in_specs=[
        _row_spec(D), _row_spec(HALF), _row_spec(HALF), _row_spec(1),
        _full_spec(HALF, D), _full_spec(HALF, D), _full_spec(D, D),
        _full_spec(1, D),
    ],
    out_specs=_row_spec(D),
    out_shape=jax.ShapeDtypeStruct((N, D), jnp.float32),
)

_final_call = pl.pallas_call(
    _final_body,
    grid=(N // BLK,),
    in_specs=[
        _row_spec(D), _row_spec(HALF), _row_spec(HALF), _row_spec(1),
        _full_spec(HALF, D), _full_spec(HALF, D), _full_spec(D, D),
        _full_spec(1, D),
        _full_spec(D, D), _full_spec(1, D),
        _full_spec(D, D), _full_spec(1, D),
    ],
    out_specs=_row_spec(D),
    out_shape=jax.ShapeDtypeStruct((N, D), jnp.float32),
)


def kernel(x, edge_index, batch,
           l0_Wl, l0_bl, l0_Wr,
           l1_Wl, l1_bl, l1_Wr,
           l2_Wl, l2_bl, l2_Wr,
           mp_W1, mp_b1, mp_W2, mp_b2):
    src = edge_index[0]
    dst = edge_index[1]
    pad = EP - E
    src_p = jnp.concatenate([src, jnp.zeros((pad,), jnp.int32)])
    dst_p = jnp.concatenate([dst, jnp.full((pad,), N, jnp.int32)])
    dstr = dst_p.reshape(TILES, NCHUNK, CHUNK)
    src2 = jnp.stack([(src_p * 2).reshape(TILES, NCHUNK, CHUNK),
                      (src_p * 2 + 1).reshape(TILES, NCHUNK, CHUNK)])
    zrow = jnp.zeros((CHUNK, HALF), jnp.float32)
    zcnt = jnp.zeros((RPT, HALF), jnp.float32)

    ones_h = jnp.zeros((CHUNK, HALF), jnp.float32).at[:, 0].set(1.0)
    cnt16 = _make_sc_cnt()(dstr, zcnt, ones_h)
    cnt = cnt16[0, :N, 0:1] + cnt16[1, :N, 0:1]

    layers = [
        (l0_Wl[:, :HALF].T, l0_Wl[:, HALF:].T, l0_bl[None, :], l0_Wr.T),
        (l1_Wl[:, :HALF].T, l1_Wl[:, HALF:].T, l1_bl[None, :], l1_Wr.T),
        (l2_Wl[:, :HALF].T, l2_Wl[:, HALF:].T, l2_bl[None, :], l2_Wr.T),
    ]

    h = x
    for i, (wl0, wl1, bl, wr) in enumerate(layers):
        h2 = h.reshape(2 * N, HALF)
        agg = _make_sc_agg()(h2, src2, dstr, zrow)
        a0 = agg[0, :N]
        a1 = agg[1, :N]
        if i < 2:
            h = _layer_call(h, a0, a1, cnt, wl0, wl1, wr, bl)
        else:
            out = _final_call(h, a0, a1, cnt, wl0, wl1, wr, bl,
                              mp_W1.T, mp_b1[None, :], mp_W2.T, mp_b2[None, :])
    return out


# speedup vs baseline: 1.0557x; 1.0557x over previous
import functools

import jax
import jax.numpy as jnp
from jax import lax
from jax.experimental import pallas as pl
from jax.experimental.pallas import tpu as pltpu
from jax.experimental.pallas import tpu_sc as plsc

N = 10000
E = 160000
D = 256
HALF = 128

TILES = 16
CHUNK = 120
NCHUNK = 84
EP = TILES * NCHUNK * CHUNK
NP = 10112
RPT = NP // TILES


def _sc_agg_body(h2, src2, dstr, zrow,
                 agg_out,
                 accf,
                 src_stage,
                 idA0, idA1, idB0, idB1,
                 rows0, rows1,
                 i0, i1, i2, i3,
                 gsem0, gsem1):
    c = lax.axis_index("c")
    s = lax.axis_index("s")

    pltpu.sync_copy(src2.at[c, s], src_stage)

    pltpu.sync_copy(zrow, rows0)
    for t in range(RPT // CHUNK):
        pltpu.sync_copy(rows0, accf.at[pl.ds(s * RPT + t * CHUNK, CHUNK)])
    pltpu.sync_copy(rows0.at[pl.ds(0, RPT % CHUNK)],
                    accf.at[pl.ds(s * RPT + RPT - RPT % CHUNK, RPT % CHUNK)])
    plsc.subcore_barrier()

    g0p = pltpu.async_copy(h2.at[src_stage.at[0]], rows0, gsem0)
    g1p = pltpu.async_copy(h2.at[src_stage.at[1]], rows1, gsem1)

    def _body(j, carry):
        q0 = j * 4
        q1 = q0 + 1
        q2 = q0 + 2
        q3 = q0 + 3
        qn = lax.min(q0 + 4, NCHUNK - 2)
        cD0 = pltpu.async_copy(dstr.at[s, q0], idA0, i0)
        cD1 = pltpu.async_copy(dstr.at[s, q1], idA1, i1)
        cD2 = pltpu.async_copy(dstr.at[s, q2], idB0, i2)
        cD3 = pltpu.async_copy(dstr.at[s, q3], idB1, i3)
        pltpu.make_async_copy(h2.at[src_stage.at[q0]], rows0, gsem0).wait()
        cD0.wait()
        pltpu.sync_copy(rows0, accf.at[idA0], add=True)
        g2 = pltpu.async_copy(h2.at[src_stage.at[q2]], rows0, gsem0)
        pltpu.make_async_copy(h2.at[src_stage.at[q1]], rows1, gsem1).wait()
        cD1.wait()
        pltpu.sync_copy(rows1, accf.at[idA1], add=True)
        g3 = pltpu.async_copy(h2.at[src_stage.at[q3]], rows1, gsem1)
        g2.wait()
        cD2.wait()
        pltpu.sync_copy(rows0, accf.at[idB0], add=True)
        pltpu.async_copy(h2.at[src_stage.at[qn]], rows0, gsem0)
        g3.wait()
        cD3.wait()
        pltpu.sync_copy(rows1, accf.at[idB1], add=True)
        pltpu.async_copy(h2.at[src_stage.at[qn + 1]], rows1, gsem1)
        return carry

    lax.fori_loop(0, NCHUNK // 4, _body, 0)

    pltpu.make_async_copy(h2.at[src_stage.at[NCHUNK - 2]], rows0, gsem0).wait()
    pltpu.make_async_copy(h2.at[src_stage.at[NCHUNK - 1]], rows1, gsem1).wait()
    plsc.subcore_barrier()

    pltpu.sync_copy(accf.at[pl.ds(s * RPT, RPT)],
                    agg_out.at[c, pl.ds(s * RPT, RPT)])


@functools.cache
def _make_sc_agg():
    return pl.kernel(
        _sc_agg_body,
        mesh=plsc.VectorSubcoreMesh(core_axis_name="c", subcore_axis_name="s"),
        out_type=jax.ShapeDtypeStruct((2, NP, HALF), jnp.float32),
        scratch_types=(
            [pltpu.VMEM_SHARED((NP, HALF), jnp.float32)]
            + [pltpu.VMEM((NCHUNK, CHUNK), jnp.int32)]
            + [pltpu.VMEM((CHUNK,), jnp.int32) for _ in range(4)]
            + [pltpu.VMEM((CHUNK, HALF), jnp.float32) for _ in range(2)]
            + [pltpu.SemaphoreType.DMA for _ in range(6)]
        ),
    )


def _sc_cnt_body(dstr, zcnt, ones_h,
                 cnt_out,
                 accc,
                 idA, idB, ones_b,
                 c0, c1):
    c = lax.axis_index("c")
    s = lax.axis_index("s")
    half = NCHUNK // 2
    base = c * half

    pltpu.async_copy(dstr.at[s, base], idA, c0)
    pltpu.async_copy(dstr.at[s, base + 1], idB, c1)
    pltpu.sync_copy(ones_h, ones_b)
    pltpu.sync_copy(zcnt, accc.at[pl.ds(s * RPT, RPT)])
    plsc.subcore_barrier()

    def _body(j, carry):
        q = base + j * 2
        qn = lax.min(q + 2, base + half - 2)
        pltpu.make_async_copy(dstr.at[s, q], idA, c0).wait()
        pltpu.sync_copy(ones_b, accc.at[idA], add=True)
        pltpu.async_copy(dstr.at[s, qn], idA, c0)
        pltpu.make_async_copy(dstr.at[s, q + 1], idB, c1).wait()
        pltpu.sync_copy(ones_b, accc.at[idB], add=True)
        pltpu.async_copy(dstr.at[s, qn + 1], idB, c1)
        return carry

    lax.fori_loop(0, NCHUNK // 2 // 2, _body, 0)
    pltpu.make_async_copy(dstr.at[s, base + half - 2], idA, c0).wait()
    pltpu.make_async_copy(dstr.at[s, base + half - 1], idB, c1).wait()
    plsc.subcore_barrier()

    pltpu.sync_copy(accc.at[pl.ds(s * RPT, RPT)],
                    cnt_out.at[c, pl.ds(s * RPT, RPT)])


@functools.cache
def _make_sc_cnt():
    return pl.kernel(
        _sc_cnt_body,
        mesh=plsc.VectorSubcoreMesh(core_axis_name="c", subcore_axis_name="s"),
        out_type=jax.ShapeDtypeStruct((2, NP, HALF), jnp.float32),
        scratch_types=[
            pltpu.VMEM_SHARED((NP, HALF), jnp.float32),
            pltpu.VMEM((CHUNK,), jnp.int32),
            pltpu.VMEM((CHUNK,), jnp.int32),
            pltpu.VMEM((CHUNK, HALF), jnp.float32),
            pltpu.SemaphoreType.DMA,
            pltpu.SemaphoreType.DMA,
        ],
    )


BLK = 1000


def _layer_body(h, a0, a1, cnt, wl0, wl1, wr, bl, o):
    inv = 1.0 / jnp.maximum(cnt[...], 1.0)
    acc = jnp.dot(a0[...] * inv, wl0[...], preferred_element_type=jnp.float32)
    acc += jnp.dot(a1[...] * inv, wl1[...], preferred_element_type=jnp.float32)
    acc += jnp.dot(h[...], wr[...], preferred_element_type=jnp.float32)
    o[...] = jnp.maximum(acc + bl[...], 0.0)


def _final_body(h, a0, a1, cnt, wl0, wl1, wr, bl, w1, b1, w2, b2, o):
    inv = 1.0 / jnp.maximum(cnt[...], 1.0)
    acc = jnp.dot(a0[...] * inv, wl0[...], preferred_element_type=jnp.float32)
    acc += jnp.dot(a1[...] * inv, wl1[...], preferred_element_type=jnp.float32)
    acc += jnp.dot(h[...], wr[...], preferred_element_type=jnp.float32)
    t = jnp.maximum(acc + bl[...], 0.0)
    u = jnp.dot(t, w1[...], preferred_element_type=jnp.float32) + b1[...]
    v = jnp.dot(u, w2[...], preferred_element_type=jnp.float32) + b2[...]
    m = jnp.max(v, axis=1, keepdims=True)
    lse = jnp.log(jnp.sum(jnp.exp(v - m), axis=1, keepdims=True)) + m
    o[...] = v - lse


def _row_spec(w):
    return pl.BlockSpec((BLK, w), lambda i: (i, 0))


def _full_spec(r, c):
    return pl.BlockSpec((r, c), lambda i: (0, 0))


_layer_call = pl.pallas_call(
    _layer_body,
    grid=(N // BLK,),
    in_specs=[
        _row_spec(D), _row_spec(HALF), _row_spec(HALF), _row_spec(1),
        _full_spec(HALF, D), _full_spec(HALF, D), _full_spec(D, D),
        _full_spec(1, D),
    ],
    out_specs=_row_spec(D),
    out_shape=jax.ShapeDtypeStruct((N, D), jnp.float32),
)

_final_call = pl.pallas_call(
    _final_body,
    grid=(N // BLK,),
    in_specs=[
        _row_spec(D), _row_spec(HALF), _row_spec(HALF), _row_spec(1),
        _full_spec(HALF, D), _full_spec(HALF, D), _full_spec(D, D),
        _full_spec(1, D),
        _full_spec(D, D), _full_spec(1, D),
        _full_spec(D, D), _full_spec(1, D),
    ],
    out_specs=_row_spec(D),
    out_shape=jax.ShapeDtypeStruct((N, D), jnp.float32),
)


def kernel(x, edge_index, batch,
           l0_Wl, l0_bl, l0_Wr,
           l1_Wl, l1_bl, l1_Wr,
           l2_Wl, l2_bl, l2_Wr,
           mp_W1, mp_b1, mp_W2, mp_b2):
    src = edge_index[0]
    dst = edge_index[1]
    pad = EP - E
    src_p = jnp.concatenate([src, jnp.zeros((pad,), jnp.int32)])
    dst_p = jnp.concatenate([dst, jnp.full((pad,), N, jnp.int32)])
    dstr = dst_p.reshape(TILES, NCHUNK, CHUNK)
    src2 = jnp.stack([(src_p * 2).reshape(TILES, NCHUNK, CHUNK),
                      (src_p * 2 + 1).reshape(TILES, NCHUNK, CHUNK)])
    zrow = jnp.zeros((CHUNK, HALF), jnp.float32)
    zcnt = jnp.zeros((RPT, HALF), jnp.float32)

    ones_h = jnp.zeros((CHUNK, HALF), jnp.float32).at[:, 0].set(1.0)
    cnt16 = _make_sc_cnt()(dstr, zcnt, ones_h)
    cnt = cnt16[0, :N, 0:1] + cnt16[1, :N, 0:1]

    layers = [
        (l0_Wl[:, :HALF].T, l0_Wl[:, HALF:].T, l0_bl[None, :], l0_Wr.T),
        (l1_Wl[:, :HALF].T, l1_Wl[:, HALF:].T, l1_bl[None, :], l1_Wr.T),
        (l2_Wl[:, :HALF].T, l2_Wl[:, HALF:].T, l2_bl[None, :], l2_Wr.T),
    ]

    h = x
    for i, (wl0, wl1, bl, wr) in enumerate(layers):
        h2 = h.reshape(2 * N, HALF)
        agg = _make_sc_agg()(h2, src2, dstr, zrow)
        a0 = agg[0, :N]
        a1 = agg[1, :N]
        if i < 2:
            h = _layer_call(h, a0, a1, cnt, wl0, wl1, wr, bl)
        else:
            out = _final_call(h, a0, a1, cnt, wl0, wl1, wr, bl,
                              mp_W1.T, mp_b1[None, :], mp_W2.T, mp_b2[None, :])
    return out


# speedup vs baseline: 1.0569x; 1.0011x over previous
import functools

import jax
import jax.numpy as jnp
from jax import lax
from jax.experimental import pallas as pl
from jax.experimental.pallas import tpu as pltpu
from jax.experimental.pallas import tpu_sc as plsc

N = 10000
E = 160000
D = 256
HALF = 128

TILES = 16
CHUNK = 120
NCHUNK = 84
EP = TILES * NCHUNK * CHUNK
NP = 10112
RPT = NP // TILES


def _sc_agg_body(h2, src2, dstr, zrow,
                 agg_out,
                 accf,
                 src_stage,
                 idA0, idA1, idB0, idB1,
                 rows0, rows1,
                 i0, i1, i2, i3,
                 gsem0, gsem1):
    c = lax.axis_index("c")
    s = lax.axis_index("s")

    pltpu.sync_copy(src2.at[c, s], src_stage)

    pltpu.sync_copy(zrow, rows0)
    for t in range(RPT // CHUNK):
        pltpu.sync_copy(rows0, accf.at[pl.ds(s * RPT + t * CHUNK, CHUNK)])
    pltpu.sync_copy(rows0.at[pl.ds(0, RPT % CHUNK)],
                    accf.at[pl.ds(s * RPT + RPT - RPT % CHUNK, RPT % CHUNK)])
    plsc.subcore_barrier()

    g0p = pltpu.async_copy(h2.at[src_stage.at[0]], rows0, gsem0)
    g1p = pltpu.async_copy(h2.at[src_stage.at[1]], rows1, gsem1)

    def _body(j, carry):
        q0 = j * 4
        q1 = q0 + 1
        q2 = q0 + 2
        q3 = q0 + 3
        qn = lax.min(q0 + 4, NCHUNK - 2)
        cD0 = pltpu.async_copy(dstr.at[s, q0], idA0, i0)
        cD1 = pltpu.async_copy(dstr.at[s, q1], idA1, i1)
        cD2 = pltpu.async_copy(dstr.at[s, q2], idB0, i2)
        cD3 = pltpu.async_copy(dstr.at[s, q3], idB1, i3)
        pltpu.make_async_copy(h2.at[src_stage.at[q0]], rows0, gsem0).wait()
        cD0.wait()
        pltpu.sync_copy(rows0, accf.at[idA0], add=True)
        g2 = pltpu.async_copy(h2.at[src_stage.at[q2]], rows0, gsem0)
        pltpu.make_async_copy(h2.at[src_stage.at[q1]], rows1, gsem1).wait()
        cD1.wait()
        pltpu.sync_copy(rows1, accf.at[idA1], add=True)
        g3 = pltpu.async_copy(h2.at[src_stage.at[q3]], rows1, gsem1)
        g2.wait()
        cD2.wait()
        pltpu.sync_copy(rows0, accf.at[idB0], add=True)
        pltpu.async_copy(h2.at[src_stage.at[qn]], rows0, gsem0)
        g3.wait()
        cD3.wait()
        pltpu.sync_copy(rows1, accf.at[idB1], add=True)
        pltpu.async_copy(h2.at[src_stage.at[qn + 1]], rows1, gsem1)
        return carry

    lax.fori_loop(0, NCHUNK // 4, _body, 0)

    pltpu.make_async_copy(h2.at[src_stage.at[NCHUNK - 2]], rows0, gsem0).wait()
    pltpu.make_async_copy(h2.at[src_stage.at[NCHUNK - 1]], rows1, gsem1).wait()
    plsc.subcore_barrier()

    pltpu.sync_copy(accf.at[pl.ds(s * RPT, RPT)],
                    agg_out.at[c, pl.ds(s * RPT, RPT)])


@functools.cache
def _make_sc_agg():
    return pl.kernel(
        _sc_agg_body,
        mesh=plsc.VectorSubcoreMesh(core_axis_name="c", subcore_axis_name="s"),
        out_type=jax.ShapeDtypeStruct((2, NP, HALF), jnp.float32),
        scratch_types=(
            [pltpu.VMEM_SHARED((NP, HALF), jnp.float32)]
            + [pltpu.VMEM((NCHUNK, CHUNK), jnp.int32)]
            + [pltpu.VMEM((CHUNK,), jnp.int32) for _ in range(4)]
            + [pltpu.VMEM((CHUNK, HALF), jnp.float32) for _ in range(2)]
            + [pltpu.SemaphoreType.DMA for _ in range(6)]
        ),
    )


def _sc_cnt_body(dstr, zcnt, ones_h,
                 cnt_out,
                 accc,
                 idA, idB, ones_b,
                 c0, c1):
    c = lax.axis_index("c")
    s = lax.axis_index("s")
    half = NCHUNK // 2
    base = c * half

    pltpu.async_copy(dstr.at[s, base], idA, c0)
    pltpu.async_copy(dstr.at[s, base + 1], idB, c1)
    pltpu.sync_copy(ones_h, ones_b)
    pltpu.sync_copy(zcnt, accc.at[pl.ds(s * RPT, RPT)])
    plsc.subcore_barrier()

    def _body(j, carry):
        q = base + j * 2
        qn = lax.min(q + 2, base + half - 2)
        pltpu.make_async_copy(dstr.at[s, q], idA, c0).wait()
        pltpu.sync_copy(ones_b, accc.at[idA], add=True)
        pltpu.async_copy(dstr.at[s, qn], idA, c0)
        pltpu.make_async_copy(dstr.at[s, q + 1], idB, c1).wait()
        pltpu.sync_copy(ones_b, accc.at[idB], add=True)
        pltpu.async_copy(dstr.at[s, qn + 1], idB, c1)
        return carry

    lax.fori_loop(0, NCHUNK // 2 // 2, _body, 0)
    pltpu.make_async_copy(dstr.at[s, base + half - 2], idA, c0).wait()
    pltpu.make_async_copy(dstr.at[s, base + half - 1], idB, c1).wait()
    plsc.subcore_barrier()

    pltpu.sync_copy(accc.at[pl.ds(s * RPT, RPT)],
                    cnt_out.at[c, pl.ds(s * RPT, RPT)])


@functools.cache
def _make_sc_cnt():
    return pl.kernel(
        _sc_cnt_body,
        mesh=plsc.VectorSubcoreMesh(core_axis_name="c", subcore_axis_name="s"),
        out_type=jax.ShapeDtypeStruct((2, NP, HALF), jnp.float32),
        scratch_types=[
            pltpu.VMEM_SHARED((NP, HALF), jnp.float32),
            pltpu.VMEM((CHUNK,), jnp.int32),
            pltpu.VMEM((CHUNK,), jnp.int32),
            pltpu.VMEM((CHUNK, HALF), jnp.float32),
            pltpu.SemaphoreType.DMA,
            pltpu.SemaphoreType.DMA,
        ],
    )


BLK = 2000


def _layer_body(h, a0, a1, cnt, wl0, wl1, wr, bl, o):
    inv = 1.0 / jnp.maximum(cnt[...], 1.0)
    acc = jnp.dot(a0[...] * inv, wl0[...], preferred_element_type=jnp.float32)
    acc += jnp.dot(a1[...] * inv, wl1[...], preferred_element_type=jnp.float32)
    acc += jnp.dot(h[...], wr[...], preferred_element_type=jnp.float32)
    o[...] = jnp.maximum(acc + bl[...], 0.0)


def _final_body(h, a0, a1, cnt, wl0, wl1, wr, bl, w1, b1, w2, b2, o):
    inv = 1.0 / jnp.maximum(cnt[...], 1.0)
    acc = jnp.dot(a0[...] * inv, wl0[...], preferred_element_type=jnp.float32)
    acc += jnp.dot(a1[...] * inv, wl1[...], preferred_element_type=jnp.float32)
    acc += jnp.dot(h[...], wr[...], preferred_element_type=jnp.float32)
    t = jnp.maximum(acc + bl[...], 0.0)
    u = jnp.dot(t, w1[...], preferred_element_type=jnp.float32) + b1[...]
    v = jnp.dot(u, w2[...], preferred_element_type=jnp.float32) + b2[...]
    m = jnp.max(v, axis=1, keepdims=True)
    lse = jnp.log(jnp.sum(jnp.exp(v - m), axis=1, keepdims=True)) + m
    o[...] = v - lse


def _row_spec(w):
    return pl.BlockSpec((BLK, w), lambda i: (i, 0))


def _full_spec(r, c):
    return pl.BlockSpec((r, c), lambda i: (0, 0))


_layer_call = pl.pallas_call(
    _layer_body,
    grid=(N // BLK,),
    in_specs=[
        _row_spec(D), _row_spec(HALF), _row_spec(HALF), _row_spec(1),
        _full_spec(HALF, D), _full_spec(HALF, D), _full_spec(D, D),
        _full_spec(1, D),
    ],
    out_specs=_row_spec(D),
    out_shape=jax.ShapeDtypeStruct((N, D), jnp.float32),
)

_final_call = pl.pallas_call(
    _final_body,
    grid=(N // BLK,),
    in_specs=[
        _row_spec(D), _row_spec(HALF), _row_spec(HALF), _row_spec(1),
        _full_spec(HALF, D), _full_spec(HALF, D), _full_spec(D, D),
        _full_spec(1, D),
        _full_spec(D, D), _full_spec(1, D),
        _full_spec(D, D), _full_spec(1, D),
    ],
    out_specs=_row_spec(D),
    out_shape=jax.ShapeDtypeStruct((N, D), jnp.float32),
)


def kernel(x, edge_index, batch,
           l0_Wl, l0_bl, l0_Wr,
           l1_Wl, l1_bl, l1_Wr,
           l2_Wl, l2_bl, l2_Wr,
           mp_W1, mp_b1, mp_W2, mp_b2):
    src = edge_index[0]
    dst = edge_index[1]
    pad = EP - E
    src_p = jnp.concatenate([src, jnp.zeros((pad,), jnp.int32)])
    dst_p = jnp.concatenate([dst, jnp.full((pad,), N, jnp.int32)])
    dstr = dst_p.reshape(TILES, NCHUNK, CHUNK)
    src2 = jnp.stack([(src_p * 2).reshape(TILES, NCHUNK, CHUNK),
                      (src_p * 2 + 1).reshape(TILES, NCHUNK, CHUNK)])
    zrow = jnp.zeros((CHUNK, HALF), jnp.float32)
    zcnt = jnp.zeros((RPT, HALF), jnp.float32)

    ones_h = jnp.zeros((CHUNK, HALF), jnp.float32).at[:, 0].set(1.0)
    cnt16 = _make_sc_cnt()(dstr, zcnt, ones_h)
    cnt = cnt16[0, :N, 0:1] + cnt16[1, :N, 0:1]

    layers = [
        (l0_Wl[:, :HALF].T, l0_Wl[:, HALF:].T, l0_bl[None, :], l0_Wr.T),
        (l1_Wl[:, :HALF].T, l1_Wl[:, HALF:].T, l1_bl[None, :], l1_Wr.T),
        (l2_Wl[:, :HALF].T, l2_Wl[:, HALF:].T, l2_bl[None, :], l2_Wr.T),
    ]

    h = x
    for i, (wl0, wl1, bl, wr) in enumerate(layers):
        h2 = h.reshape(2 * N, HALF)
        agg = _make_sc_agg()(h2, src2, dstr, zrow)
        a0 = agg[0, :N]
        a1 = agg[1, :N]
        if i < 2:
            h = _layer_call(h, a0, a1, cnt, wl0, wl1, wr, bl)
        else:
            out = _final_call(h, a0, a1, cnt, wl0, wl1, wr, bl,
                              mp_W1.T, mp_b1[None, :], mp_W2.T, mp_b2[None, :])
    return out
